# baseline (speedup 1.0000x reference)
import functools

import jax
import jax.numpy as jnp
from jax import lax
from jax.experimental import pallas as pl
from jax.experimental.pallas import tpu as pltpu
from jax.experimental.pallas import tpu_sc as plsc

_N = 10000
_E = 160000
_D = 256
_DH = 128
_NT = 16
_NC = 2
_NB = 80
_B = 128
_EPAD = _NT * _NB * _B
_B2 = 64
_NB2 = _NB * _B // _B2
_AR = 10240
_RPT = _AR // _NT
_DUMMY = _N
_RB = 2000


def _dot_t(a, b):
    return lax.dot_general(a, b, (((1,), (1,)), ((), ())),
                           preferred_element_type=jnp.float32)


@functools.lru_cache(maxsize=None)
def _build():
    f32 = jnp.float32
    mesh = plsc.VectorSubcoreMesh(core_axis_name="c", subcore_axis_name="s")

    @functools.partial(
        pl.kernel,
        out_type=jax.ShapeDtypeStruct((_NC * _AR, _DH), f32),
        mesh=mesh,
        scratch_types=[
            pltpu.VMEM_SHARED((_AR, _DH), f32),
            pltpu.VMEM((_NB, _B), jnp.int32),
            pltpu.VMEM((_B, _DH), f32),
            pltpu.VMEM((64, _DH), f32),
            pltpu.SemaphoreType.DMA,
        ],
    )
    def deg_kernel(row_h, ones_h, z_h, out_h, sdeg, row_v, ones_v, zb_v, sem):
        c = lax.axis_index("c")
        s = lax.axis_index("s")
        pltpu.sync_copy(row_h.at[s], row_v)
        pltpu.sync_copy(ones_h, ones_v)
        pltpu.sync_copy(z_h, zb_v)
        base = s * _RPT
        for j in range(_RPT // 64):
            pltpu.sync_copy(zb_v, sdeg.at[pl.ds(base + j * 64, 64)])
        plsc.subcore_barrier()

        half = _NB // _NC
        wave = 8

        def body(t, carry):
            for w in range(wave):
                b = c * half + t * wave + w
                pltpu.async_copy(ones_v, sdeg.at[row_v.at[b]], sem, add=True)
            for w in range(wave):
                pltpu.make_async_copy(ones_v, sdeg.at[row_v.at[0]],
                                      sem).wait()
            return carry

        lax.fori_loop(0, half // wave, body, 0)
        plsc.subcore_barrier()
        for j in range(_RPT // _B):
            pltpu.sync_copy(sdeg.at[pl.ds(base + j * _B, _B)], ones_v)
            pltpu.sync_copy(
                ones_v, out_h.at[pl.ds(c * _AR + base + j * _B, _B)])

    @functools.partial(
        pl.kernel,
        out_type=(jax.ShapeDtypeStruct((_AR, _DH), f32),
                  jax.ShapeDtypeStruct((_AR, _DH), f32)),
        mesh=mesh,
        scratch_types=[
            pltpu.VMEM_SHARED((_AR, _DH), f32),
            pltpu.VMEM((_NB, _B), jnp.int32),
            pltpu.VMEM((_NB, _B), jnp.int32),
            pltpu.VMEM((_B2, _DH), f32),
            pltpu.VMEM((_B2, _DH), f32),
            pltpu.VMEM((_B2, _DH), f32),
            pltpu.SemaphoreType.DMA,
            pltpu.SemaphoreType.DMA,
            pltpu.SemaphoreType.DMA,
            pltpu.SemaphoreType.DMA,
            pltpu.SemaphoreType.DMA,
            pltpu.SemaphoreType.DMA,
        ],
    )
    def hop_kernel(ta_h, tb_h, row_h, col_h, z_h, ya_h, yb_h,
                   acc, row_v, col_v, g0, g1, g2,
                   gs0, gs1, gs2, ss0, ss1, ss2):
        c = lax.axis_index("c")
        s = lax.axis_index("s")
        pltpu.sync_copy(row_h.at[s], row_v)
        pltpu.sync_copy(col_h.at[s], col_v)
        pltpu.sync_copy(z_h, g0)
        base = s * _RPT
        for j in range(_RPT // _B2):
            pltpu.async_copy(g0, acc.at[pl.ds(base + j * _B2, _B2)], ss0)
        for j in range(_RPT // _B2):
            pltpu.make_async_copy(g0, acc.at[pl.ds(base, _B2)], ss0).wait()
        plsc.subcore_barrier()

        BUF = (g0, g1, g2)
        GS = (gs0, gs1, gs2)
        SS = (ss0, ss1, ss2)

        def run(table_h, out_h):
            def start_g(b, off, buf, sem):
                pltpu.async_copy(
                    table_h.at[col_v.at[b, pl.ds(off, _B2)]], buf, sem)

            def wait_g(buf, sem):
                pltpu.make_async_copy(
                    table_h.at[col_v.at[0, pl.ds(0, _B2)]], buf, sem).wait()

            def start_s(b, off, buf, sem):
                pltpu.async_copy(
                    buf, acc.at[row_v.at[b, pl.ds(off, _B2)]], sem, add=True)

            def wait_s(sem):
                pltpu.make_async_copy(
                    g0, acc.at[row_v.at[0, pl.ds(0, _B2)]], sem).wait()

            start_g(0, 0, g0, gs0)
            start_g(0, _B2, g1, gs1)
            start_g(1, 0, g2, gs2)

            def round_body(i, carry):
                for k in range(6):
                    j = k % 3
                    b = 3 * i + k // 2
                    off = (k % 2) * _B2
                    b2 = 3 * i + (k + 3) // 2
                    off2 = ((k + 1) % 2) * _B2
                    wait_g(BUF[j], GS[j])
                    start_s(b, off, BUF[j], SS[j])
                    wait_s(SS[j])
                    start_g(b2, off2, BUF[j], GS[j])
                return carry

            lax.fori_loop(0, (_NB * 2 - 4) // 6, round_body, 0)
            wait_g(g0, gs0)
            start_s(_NB - 2, 0, g0, ss0)
            wait_s(ss0)
            start_g(_NB - 1, _B2, g0, gs0)
            wait_g(g1, gs1)
            start_s(_NB - 2, _B2, g1, ss1)
            wait_s(ss1)
            wait_g(g2, gs2)
            start_s(_NB - 1, 0, g2, ss2)
            wait_s(ss2)
            wait_g(g0, gs0)
            start_s(_NB - 1, _B2, g0, ss0)
            wait_s(ss0)
            plsc.subcore_barrier()

            def ro_src(j):
                return acc.at[pl.ds(base + j * _B2, _B2)]

            def ro_dst(j):
                return out_h.at[pl.ds(base + j * _B2, _B2)]

            nro = _RPT // _B2
            pltpu.async_copy(ro_src(0), g0, gs0)
            for j in range(nro):
                k = j % 3
                if j + 1 < nro:
                    k1 = (j + 1) % 3
                    if j + 1 >= 3:
                        pltpu.make_async_copy(BUF[k1], ro_dst(0),
                                              SS[k1]).wait()
                    pltpu.async_copy(ro_src(j + 1), BUF[k1], GS[k1])
                pltpu.make_async_copy(ro_src(0), BUF[k], GS[k]).wait()
                pltpu.async_copy(BUF[k], ro_dst(j), SS[k])
            for k in ((nro - 3) % 3, (nro - 2) % 3, (nro - 1) % 3):
                pltpu.make_async_copy(BUF[k], ro_dst(0), SS[k]).wait()

        @pl.when(c == 0)
        def _run_a():
            run(ta_h, ya_h)

        @pl.when(c == 1)
        def _run_b():
            run(tb_h, yb_h)

    def prep_body(p_ref, s_ref, s2_ref):
        d = jnp.sum(p_ref[...], axis=0, keepdims=True) + 1.0
        s_ref[...] = lax.rsqrt(d)
        s2_ref[...] = 1.0 / d

    prep = pl.pallas_call(
        prep_body,
        out_shape=(jax.ShapeDtypeStruct((1, _AR), f32),
                   jax.ShapeDtypeStruct((1, _AR), f32)),
    )

    def mm_body(x_ref, w_ref, ta_ref, tb_ref):
        t = _dot_t(x_ref[...], w_ref[...])
        ta_ref[...] = t[:, :_DH]
        tb_ref[...] = t[:, _DH:]

    mm = pl.pallas_call(
        mm_body,
        grid=(_N // _RB,),
        in_specs=[
            pl.BlockSpec((_RB, _D), lambda i: (i, 0)),
            pl.BlockSpec((_D, _D), lambda i: (0, 0)),
        ],
        out_specs=(pl.BlockSpec((_RB, _DH), lambda i: (i, 0)),
                   pl.BlockSpec((_RB, _DH), lambda i: (i, 0))),
        out_shape=(jax.ShapeDtypeStruct((_N, _DH), f32),
                   jax.ShapeDtypeStruct((_N, _DH), f32)),
    )

    def scale_body(xa_ref, xb_ref, s_ref, ta_ref, tb_ref):
        ta_ref[...] = xa_ref[...] * s_ref[...]
        tb_ref[...] = xb_ref[...] * s_ref[...]

    scale = pl.pallas_call(
        scale_body,
        grid=(_N // _RB,),
        in_specs=[
            pl.BlockSpec((_RB, _DH), lambda i: (i, 0)),
            pl.BlockSpec((_RB, _DH), lambda i: (i, 0)),
            pl.BlockSpec((_RB, 1), lambda i: (i, 0)),
        ],
        out_specs=(pl.BlockSpec((_RB, _DH), lambda i: (i, 0)),
                   pl.BlockSpec((_RB, _DH), lambda i: (i, 0))),
        out_shape=(jax.ShapeDtypeStruct((_N, _DH), f32),
                   jax.ShapeDtypeStruct((_N, _DH), f32)),
    )

    def mid_body(ya_ref, yb_ref, ta_ref, tb_ref, s2_ref, oa_ref, ob_ref):
        oa_ref[...] = (ya_ref[...] + ta_ref[...]) * s2_ref[...]
        ob_ref[...] = (yb_ref[...] + tb_ref[...]) * s2_ref[...]

    half_spec = pl.BlockSpec((_RB, _DH), lambda i: (i, 0))
    col_spec = pl.BlockSpec((_RB, 1), lambda i: (i, 0))
    mid = pl.pallas_call(
        mid_body,
        grid=(_N // _RB,),
        in_specs=[half_spec, half_spec, half_spec, half_spec, col_spec],
        out_specs=(half_spec, half_spec),
        out_shape=(jax.ShapeDtypeStruct((_N, _DH), f32),
                   jax.ShapeDtypeStruct((_N, _DH), f32)),
    )

    def fin_body(ya_ref, yb_ref, ta_ref, tb_ref, s_ref, o_ref):
        a = (ya_ref[...] + ta_ref[...]) * s_ref[...]
        b = (yb_ref[...] + tb_ref[...]) * s_ref[...]
        o_ref[...] = jnp.concatenate([a, b], axis=1)

    fin = pl.pallas_call(
        fin_body,
        grid=(_N // _RB,),
        in_specs=[half_spec, half_spec, half_spec, half_spec, col_spec],
        out_specs=pl.BlockSpec((_RB, _D), lambda i: (i, 0)),
        out_shape=jax.ShapeDtypeStruct((_N, _D), f32),
    )

    return {"deg": deg_kernel, "hop": hop_kernel, "prep": prep,
            "mm": mm, "scale": scale, "mid": mid, "fin": fin}


def kernel(x, edge_index, W):
    fns = _build()
    row = edge_index[0].astype(jnp.int32)
    col = edge_index[1].astype(jnp.int32)
    pad = _EPAD - _E
    row_p = jnp.concatenate(
        [row, jnp.full((pad,), _DUMMY, jnp.int32)]).reshape(_NT, _NB, _B)
    col_p = jnp.concatenate(
        [col, jnp.zeros((pad,), jnp.int32)]).reshape(_NT, _NB, _B)
    ones_b = jnp.ones((_B, _DH), jnp.float32)
    z64 = jnp.zeros((64, _DH), jnp.float32)

    xwa, xwb = fns["mm"](x, W)
    degp = fns["deg"](row_p, ones_b, z64)
    parts = degp[:, 0].reshape(_NC, _AR)
    s_row, s2_row = fns["prep"](parts)
    s_col = s_row.reshape(_AR, 1)[:_N]
    s2_col = s2_row.reshape(_AR, 1)[:_N]

    t0a, t0b = fns["scale"](xwa, xwb, s_col)
    y1a, y1b = fns["hop"](t0a, t0b, row_p, col_p, z64)
    t2a, t2b = fns["mid"](y1a[:_N], y1b[:_N], t0a, t0b, s2_col)
    y2a, y2b = fns["hop"](t2a, t2b, row_p, col_p, z64)
    return fns["fin"](y2a[:_N], y2b[:_N], t2a, t2b, s_col)


# speedup vs baseline: 1.0144x; 1.0144x over previous
import functools

import jax
import jax.numpy as jnp
from jax import lax
from jax.experimental import pallas as pl
from jax.experimental.pallas import tpu as pltpu
from jax.experimental.pallas import tpu_sc as plsc

_N = 10000
_E = 160000
_D = 256
_DH = 128
_NT = 16
_NC = 2
_NB = 80
_B = 128
_EPAD = _NT * _NB * _B
_B2 = 64
_NB2 = _NB * _B // _B2
_AR = 10240
_RPT = _AR // _NT
_DUMMY = _N
_RB = 2000


def _dot_t(a, b):
    return lax.dot_general(a, b, (((1,), (1,)), ((), ())),
                           preferred_element_type=jnp.float32)


@functools.lru_cache(maxsize=None)
def _build():
    f32 = jnp.float32
    mesh = plsc.VectorSubcoreMesh(core_axis_name="c", subcore_axis_name="s")

    @functools.partial(
        pl.kernel,
        out_type=jax.ShapeDtypeStruct((_NC * _AR, _DH), f32),
        mesh=mesh,
        scratch_types=[
            pltpu.VMEM_SHARED((_AR, _DH), f32),
            pltpu.VMEM((_NB, _B), jnp.int32),
            pltpu.VMEM((_B, _DH), f32),
            pltpu.VMEM((64, _DH), f32),
            pltpu.SemaphoreType.DMA,
        ],
    )
    def deg_kernel(row_h, ones_h, z_h, out_h, sdeg, row_v, ones_v, zb_v, sem):
        c = lax.axis_index("c")
        s = lax.axis_index("s")
        pltpu.sync_copy(row_h.at[s], row_v)
        pltpu.sync_copy(ones_h, ones_v)
        pltpu.sync_copy(z_h, zb_v)
        base = s * _RPT
        for j in range(_RPT // 64):
            pltpu.sync_copy(zb_v, sdeg.at[pl.ds(base + j * 64, 64)])
        plsc.subcore_barrier()

        half = _NB // _NC
        wave = 8

        def body(t, carry):
            for w in range(wave):
                b = c * half + t * wave + w
                pltpu.async_copy(ones_v, sdeg.at[row_v.at[b]], sem, add=True)
            for w in range(wave):
                pltpu.make_async_copy(ones_v, sdeg.at[row_v.at[0]],
                                      sem).wait()
            return carry

        lax.fori_loop(0, half // wave, body, 0)
        plsc.subcore_barrier()
        for j in range(_RPT // _B):
            pltpu.sync_copy(sdeg.at[pl.ds(base + j * _B, _B)], ones_v)
            pltpu.sync_copy(
                ones_v, out_h.at[pl.ds(c * _AR + base + j * _B, _B)])

    @functools.partial(
        pl.kernel,
        out_type=(jax.ShapeDtypeStruct((_AR, _DH), f32),
                  jax.ShapeDtypeStruct((_AR, _DH), f32)),
        mesh=mesh,
        scratch_types=[
            pltpu.VMEM_SHARED((_AR, _DH), f32),
            pltpu.VMEM((_NB, _B), jnp.int32),
            pltpu.VMEM((_NB, _B), jnp.int32),
            pltpu.VMEM((_B2, _DH), f32),
            pltpu.VMEM((_B2, _DH), f32),
            pltpu.VMEM((_B2, _DH), f32),
            pltpu.SemaphoreType.DMA,
            pltpu.SemaphoreType.DMA,
            pltpu.SemaphoreType.DMA,
            pltpu.SemaphoreType.DMA,
            pltpu.SemaphoreType.DMA,
            pltpu.SemaphoreType.DMA,
        ],
    )
    def hop_kernel(ta_h, tb_h, row_h, col_h, z_h, ya_h, yb_h,
                   acc, row_v, col_v, g0, g1, g2,
                   gs0, gs1, gs2, ss0, ss1, ss2):
        c = lax.axis_index("c")
        s = lax.axis_index("s")
        pltpu.sync_copy(row_h.at[s], row_v)
        pltpu.sync_copy(col_h.at[s], col_v)
        pltpu.sync_copy(z_h, g0)
        base = s * _RPT
        for j in range(_RPT // _B2):
            pltpu.async_copy(g0, acc.at[pl.ds(base + j * _B2, _B2)], ss0)
        for j in range(_RPT // _B2):
            pltpu.make_async_copy(g0, acc.at[pl.ds(base, _B2)], ss0).wait()
        plsc.subcore_barrier()

        BUF = (g0, g1, g2)
        GS = (gs0, gs1, gs2)
        SS = (ss0, ss1, ss2)

        def run(table_h, out_h):
            def start_g(b, off, buf, sem):
                pltpu.async_copy(
                    table_h.at[col_v.at[b, pl.ds(off, _B2)]], buf, sem)

            def wait_g(buf, sem):
                pltpu.make_async_copy(
                    table_h.at[col_v.at[0, pl.ds(0, _B2)]], buf, sem).wait()

            def start_s(b, off, buf, sem):
                pltpu.async_copy(
                    buf, acc.at[row_v.at[b, pl.ds(off, _B2)]], sem, add=True)

            def wait_s(sem):
                pltpu.make_async_copy(
                    g0, acc.at[row_v.at[0, pl.ds(0, _B2)]], sem).wait()

            start_g(0, 0, g0, gs0)
            start_g(0, _B2, g1, gs1)
            start_g(1, 0, g2, gs2)

            def round_body(i, carry):
                for k in range(6):
                    j = k % 3
                    b = 3 * i + k // 2
                    off = (k % 2) * _B2
                    b2 = 3 * i + (k + 3) // 2
                    off2 = ((k + 1) % 2) * _B2
                    wait_g(BUF[j], GS[j])
                    start_s(b, off, BUF[j], SS[j])
                    wait_s(SS[j])
                    start_g(b2, off2, BUF[j], GS[j])
                return carry

            lax.fori_loop(0, (_NB * 2 - 4) // 6, round_body, 0)
            wait_g(g0, gs0)
            start_s(_NB - 2, 0, g0, ss0)
            wait_s(ss0)
            start_g(_NB - 1, _B2, g0, gs0)
            wait_g(g1, gs1)
            start_s(_NB - 2, _B2, g1, ss1)
            wait_s(ss1)
            wait_g(g2, gs2)
            start_s(_NB - 1, 0, g2, ss2)
            wait_s(ss2)
            wait_g(g0, gs0)
            start_s(_NB - 1, _B2, g0, ss0)
            wait_s(ss0)
            plsc.subcore_barrier()

            def ro_src(j):
                return acc.at[pl.ds(base + j * _B2, _B2)]

            def ro_dst(j):
                return out_h.at[pl.ds(base + j * _B2, _B2)]

            nro = _RPT // _B2
            pltpu.async_copy(ro_src(0), g0, gs0)
            for j in range(nro):
                k = j % 3
                if j + 1 < nro:
                    k1 = (j + 1) % 3
                    if j + 1 >= 3:
                        pltpu.make_async_copy(BUF[k1], ro_dst(0),
                                              SS[k1]).wait()
                    pltpu.async_copy(ro_src(j + 1), BUF[k1], GS[k1])
                pltpu.make_async_copy(ro_src(0), BUF[k], GS[k]).wait()
                pltpu.async_copy(BUF[k], ro_dst(j), SS[k])
            for k in ((nro - 3) % 3, (nro - 2) % 3, (nro - 1) % 3):
                pltpu.make_async_copy(BUF[k], ro_dst(0), SS[k]).wait()

        @pl.when(c == 0)
        def _run_a():
            run(ta_h, ya_h)

        @pl.when(c == 1)
        def _run_b():
            run(tb_h, yb_h)

    def prep_body(p_ref, s_ref, s2_ref):
        d = jnp.sum(p_ref[...], axis=0, keepdims=True) + 1.0
        s_ref[...] = lax.rsqrt(d)
        s2_ref[...] = 1.0 / d

    prep = pl.pallas_call(
        prep_body,
        out_shape=(jax.ShapeDtypeStruct((1, _AR), f32),
                   jax.ShapeDtypeStruct((1, _AR), f32)),
    )

    def lin_body(x_ref, w_ref, s_ref, ta_ref, tb_ref):
        t = _dot_t(x_ref[...], w_ref[...]) * s_ref[...]
        ta_ref[...] = t[:, :_DH]
        tb_ref[...] = t[:, _DH:]

    lin = pl.pallas_call(
        lin_body,
        grid=(_N // _RB,),
        in_specs=[
            pl.BlockSpec((_RB, _D), lambda i: (i, 0)),
            pl.BlockSpec((_D, _D), lambda i: (0, 0)),
            pl.BlockSpec((_RB, 1), lambda i: (i, 0)),
        ],
        out_specs=(pl.BlockSpec((_RB, _DH), lambda i: (i, 0)),
                   pl.BlockSpec((_RB, _DH), lambda i: (i, 0))),
        out_shape=(jax.ShapeDtypeStruct((_N, _DH), f32),
                   jax.ShapeDtypeStruct((_N, _DH), f32)),
    )

    def mid_body(ya_ref, yb_ref, ta_ref, tb_ref, s2_ref, oa_ref, ob_ref):
        oa_ref[...] = (ya_ref[...] + ta_ref[...]) * s2_ref[...]
        ob_ref[...] = (yb_ref[...] + tb_ref[...]) * s2_ref[...]

    half_spec = pl.BlockSpec((_RB, _DH), lambda i: (i, 0))
    col_spec = pl.BlockSpec((_RB, 1), lambda i: (i, 0))
    mid = pl.pallas_call(
        mid_body,
        grid=(_N // _RB,),
        in_specs=[half_spec, half_spec, half_spec, half_spec, col_spec],
        out_specs=(half_spec, half_spec),
        out_shape=(jax.ShapeDtypeStruct((_N, _DH), f32),
                   jax.ShapeDtypeStruct((_N, _DH), f32)),
    )

    def fin_body(ya_ref, yb_ref, ta_ref, tb_ref, s_ref, o_ref):
        a = (ya_ref[...] + ta_ref[...]) * s_ref[...]
        b = (yb_ref[...] + tb_ref[...]) * s_ref[...]
        o_ref[...] = jnp.concatenate([a, b], axis=1)

    fin = pl.pallas_call(
        fin_body,
        grid=(_N // _RB,),
        in_specs=[half_spec, half_spec, half_spec, half_spec, col_spec],
        out_specs=pl.BlockSpec((_RB, _D), lambda i: (i, 0)),
        out_shape=jax.ShapeDtypeStruct((_N, _D), f32),
    )

    return {"deg": deg_kernel, "hop": hop_kernel, "prep": prep,
            "lin": lin, "mid": mid, "fin": fin}


def kernel(x, edge_index, W):
    fns = _build()
    row = edge_index[0].astype(jnp.int32)
    col = edge_index[1].astype(jnp.int32)
    pad = _EPAD - _E
    row_p = jnp.concatenate(
        [row, jnp.full((pad,), _DUMMY, jnp.int32)]).reshape(_NT, _NB, _B)
    col_p = jnp.concatenate(
        [col, jnp.zeros((pad,), jnp.int32)]).reshape(_NT, _NB, _B)
    ones_b = jnp.ones((_B, _DH), jnp.float32)
    z64 = jnp.zeros((64, _DH), jnp.float32)

    degp = fns["deg"](row_p, ones_b, z64)
    parts = degp[:, 0].reshape(_NC, _AR)
    s_row, s2_row = fns["prep"](parts)
    s_col = s_row.reshape(_AR, 1)[:_N]
    s2_col = s2_row.reshape(_AR, 1)[:_N]

    t0a, t0b = fns["lin"](x, W, s_col)
    y1a, y1b = fns["hop"](t0a, t0b, row_p, col_p, z64)
    t2a, t2b = fns["mid"](y1a[:_N], y1b[:_N], t0a, t0b, s2_col)
    y2a, y2b = fns["hop"](t2a, t2b, row_p, col_p, z64)
    return fns["fin"](y2a[:_N], y2b[:_N], t2a, t2b, s_col)
